# baseline (speedup 1.0000x reference)
import functools
import math

import jax
import jax.numpy as jnp
from jax import lax
from jax.experimental import pallas as pl
from jax.experimental.pallas import tpu as pltpu
from jax.experimental.pallas import tpu_sc as plsc

_NUM_ITEMS = 1000000
_D = 32
_B = 16384
_R = 2.0
_T = 1.0
_CLAMP = 1.0 + 1e-7
_FD_SCALE = math.exp(-_R / _T)

_NC = 2
_NS = 16
_NW = _NC * _NS
_BPW = _B // _NW
_CHUNK = 128
_NCHUNK = _BPW // _CHUNK
_L = 16
_NGROUP = _BPW // _L


def _sc_body(u_hbm, v_hbm, theta_hbm, out_hbm,
             idx_u, idx_v, rows_u, rows_v, out_v, sem):
    wid = lax.axis_index("s") * _NC + lax.axis_index("c")

    pltpu.sync_copy(u_hbm.at[wid], idx_u)
    pltpu.sync_copy(v_hbm.at[wid], idx_v)

    copies = []
    for j in range(_NCHUNK):
        copies.append(pltpu.async_copy(
            theta_hbm.at[idx_u.at[j]], rows_u.at[pl.ds(j * _CHUNK, _CHUNK)], sem))
        copies.append(pltpu.async_copy(
            theta_hbm.at[idx_v.at[j]], rows_v.at[pl.ds(j * _CHUNK, _CHUNK)], sem))
    for c in copies:
        c.wait()

    lane = lax.iota(jnp.int32, _L)

    def group(g, carry):
        bvec = g * _L + lane

        def dot_at(d):
            dvec = jnp.full((_L,), d, jnp.int32)
            pu = plsc.load_gather(rows_u, [bvec, dvec])
            pv = plsc.load_gather(rows_v, [bvec, dvec])
            return pu * pv

        p0 = dot_at(0)
        acc = dot_at(1)
        for d in range(2, _D):
            acc = acc + dot_at(d)

        z = jnp.maximum(p0 - acc, _CLAMP)
        w = (z - 1.0) * (z + 1.0)
        wi = lax.bitcast_convert_type(w, jnp.int32)
        y = lax.bitcast_convert_type(0x5F3759DF - (wi >> 1), jnp.float32)
        y = y * (1.5 - 0.5 * w * y * y)
        y = y * (1.5 - 0.5 * w * y * y)
        y = y * (1.5 - 0.5 * w * y * y)
        s = w * y
        fz = 1.0 / ((z + s) * _FD_SCALE + 1.0)
        out_v[pl.ds(g * _L, _L)] = fz
        return carry

    lax.fori_loop(0, _NGROUP, group, 0)
    pltpu.sync_copy(out_v, out_hbm.at[wid])


@functools.partial(jax.jit, static_argnums=())
def _sc_call(u3, v3, theta):
    mesh = plsc.VectorSubcoreMesh(core_axis_name="c", subcore_axis_name="s")
    f = pl.kernel(
        _sc_body,
        mesh=mesh,
        compiler_params=pltpu.CompilerParams(
            needs_layout_passes=False, use_tc_tiling_on_sc=False),
        out_type=jax.ShapeDtypeStruct((_NW, _BPW), jnp.float32),
        scratch_types=[
            pltpu.VMEM((_NCHUNK, _CHUNK), jnp.int32),
            pltpu.VMEM((_NCHUNK, _CHUNK), jnp.int32),
            pltpu.VMEM((_BPW, _D), jnp.float32),
            pltpu.VMEM((_BPW, _D), jnp.float32),
            pltpu.VMEM((_BPW,), jnp.float32),
            pltpu.SemaphoreType.DMA,
        ],
    )
    return f(u3, v3, theta)


def kernel(u, v, theta):
    u3 = u.astype(jnp.int32).reshape(_NW, _NCHUNK, _CHUNK)
    v3 = v.astype(jnp.int32).reshape(_NW, _NCHUNK, _CHUNK)
    out = _sc_call(u3, v3, theta.astype(jnp.float32))
    return out.reshape(_B)


# speedup vs baseline: 2.3618x; 2.3618x over previous
import functools
import math

import jax
import jax.numpy as jnp
from jax import lax
from jax.experimental import pallas as pl
from jax.experimental.pallas import tpu as pltpu
from jax.experimental.pallas import tpu_sc as plsc

_N = 1000000
_D = 32
_B = 16384
_CLAMP = 1.0 + 1e-7
_FD = math.exp(-2.0)

_NW = 32
_CHI = 1024
_NCH_G = 977
_LAST_G = 976
_CPW = 31
_RNG = _CPW * _CHI
_L = 16
_CAP = 16416
_STAG = 16896


def _c1(u2, v2, thT, th_tail, stu, stv,
        piece, bufA, srt_u, srt_v, slab, sbuf, idxb, offs, sem):
    wid = lax.axis_index("s") * 2 + lax.axis_index("c")
    base = wid * _RNG
    lane = lax.iota(jnp.int32, _L)

    def coarse(src_hbm, dst_list):
        def piece_loop(p, cnt):
            pltpu.sync_copy(src_hbm.at[pl.ds(p * 8, 8)], piece)

            def vreg_loop(k, cnt):
                val = piece[k // 8, pl.ds((k % 8) * _L, _L)]
                slot = p * 1024 + k * _L + lane
                loc = val - base
                m = (loc >= 0) & (loc < _RNG)
                packed = (loc << 14) | slot
                plsc.store_compressed(dst_list.at[pl.ds(cnt, _L)], packed, mask=m)
                c = plsc.all_reduce_population_count(m)
                return cnt + c[0]

            return lax.fori_loop(0, 64, vreg_loop, cnt)

        return lax.fori_loop(0, 16, piece_loop, jnp.int32(0))

    def bucket(cnt, dst, obase):
        nv = (cnt + _L - 1) // _L

        def pass_loop(cl, scnt):
            offs[obase + cl] = scnt

            def vl(k, scnt):
                e = bufA[pl.ds(k * _L, _L)]
                valid = (k * _L + lane) < cnt
                cid = e >> 24
                m = valid & (cid == cl)
                plsc.store_compressed(dst.at[pl.ds(scnt, _L)], e, mask=m)
                c = plsc.all_reduce_population_count(m)
                return scnt + c[0]

            return lax.fori_loop(0, nv, vl, scnt)

        total = lax.fori_loop(0, _CPW, pass_loop, jnp.int32(0))
        offs[obase + _CPW] = total

    cnt_u = coarse(u2, bufA)
    bucket(cnt_u, srt_u, 0)
    cnt_v = coarse(v2, bufA)
    bucket(cnt_v, srt_v, 33)

    my_nch = jnp.minimum(_CPW, _NCH_G - wid * _CPW)

    def chunk_loop(cl, carry):
        cg = wid * _CPW + cl

        @pl.when(cg != _LAST_G)
        def _():
            cps = []
            for g in range(4):
                for j in range(8):
                    off = pl.multiple_of(cg * _CHI + j * 128, 128)
                    cps.append(pltpu.async_copy(
                        thT.at[pl.ds(8 * g, 8), pl.ds(off, 128)],
                        slab.at[g * 8 + j], sem))
            for c in cps:
                c.wait()

        @pl.when(cg == _LAST_G)
        def _():
            cps = []
            for g in range(4):
                for j in range(4):
                    off = _LAST_G * _CHI + j * 128
                    cps.append(pltpu.async_copy(
                        thT.at[pl.ds(8 * g, 8), pl.ds(off, 128)],
                        slab.at[g * 8 + j], sem))
                cps.append(pltpu.async_copy(
                    th_tail.at[pl.ds(8 * g, 8)], slab.at[g * 8 + 4], sem))
            for c in cps:
                c.wait()

        def service(srt, obase, stag):
            o0 = offs[obase + cl]
            o1 = offs[obase + cl + 1]
            n = o1 - o0
            nv = (n + _L - 1) // _L

            def vl(k, carry):
                e = srt[pl.ds(o0 + k * _L, _L)]
                valid = (k * _L + lane) < n
                loc = e >> 14
                slot = e & 0x3FFF
                lloc = jnp.clip(loc - cl * _CHI, 0, _CHI - 1)
                q = lloc >> 7
                ll = lloc & 127
                for s in range(_D):
                    vals = plsc.load_gather(
                        slab, [(s // 8) * 8 + q,
                               jnp.full((_L,), s % 8, jnp.int32), ll])
                    plsc.store_scatter(
                        sbuf, [lane, jnp.full((_L,), s, jnp.int32)], vals)
                idxb[...] = jnp.where(valid, slot, _B + wid * _L + lane)
                pltpu.sync_copy(sbuf, stag.at[idxb])
                return carry

            lax.fori_loop(0, nv, vl, 0)

        service(srt_u, 0, stu)
        service(srt_v, 33, stv)
        return carry

    lax.fori_loop(0, my_nch, chunk_loop, 0)


@jax.jit
def _call1(u2, v2, thT, th_tail):
    mesh = plsc.VectorSubcoreMesh(core_axis_name="c", subcore_axis_name="s")
    f = pl.kernel(
        _c1,
        mesh=mesh,
        compiler_params=pltpu.CompilerParams(needs_layout_passes=False),
        out_type=[
            jax.ShapeDtypeStruct((_STAG, 128), jnp.float32),
            jax.ShapeDtypeStruct((_STAG, 128), jnp.float32),
        ],
        scratch_types=[
            pltpu.VMEM((8, 128), jnp.int32),
            pltpu.VMEM((_CAP,), jnp.int32),
            pltpu.VMEM((_CAP,), jnp.int32),
            pltpu.VMEM((_CAP,), jnp.int32),
            pltpu.VMEM((32, 8, 128), jnp.float32),
            pltpu.VMEM((_L, 128), jnp.float32),
            pltpu.VMEM((_L,), jnp.int32),
            pltpu.SMEM((70,), jnp.int32),
            pltpu.SemaphoreType.DMA,
        ],
    )
    return f(u2, v2, thT, th_tail)


def _c2(su, sv, o_ref):
    m = su[...] * sv[...]
    col = lax.broadcasted_iota(jnp.int32, (1, 128), 1)
    coeff = jnp.where(col == 0, 1.0,
                      jnp.where(col < _D, -1.0, 0.0)).astype(jnp.float32)
    z = jnp.sum(m * coeff, axis=1)
    z = jnp.maximum(z, _CLAMP)
    w = (z - 1.0) * (z + 1.0)
    s = jnp.sqrt(w)
    o_ref[...] = 1.0 / ((z + s) * _FD + 1.0)


@jax.jit
def _call2(stu, stv):
    return pl.pallas_call(
        _c2,
        grid=(_STAG // 512,),
        in_specs=[
            pl.BlockSpec((512, 128), lambda i: (i, 0)),
            pl.BlockSpec((512, 128), lambda i: (i, 0)),
        ],
        out_specs=pl.BlockSpec((512,), lambda i: (i,)),
        out_shape=jax.ShapeDtypeStruct((_STAG,), jnp.float32),
    )(stu, stv)


def kernel(u, v, theta):
    u2 = u.astype(jnp.int32).reshape(128, 128)
    v2 = v.astype(jnp.int32).reshape(128, 128)
    thT = theta.astype(jnp.float32).T
    th_tail = jnp.pad(thT[:, _LAST_G * _CHI + 512:], ((0, 0), (0, 64)))
    stu, stv = _call1(u2, v2, thT, th_tail)
    out = _call2(stu, stv)
    return out[:_B]
